# baseline (speedup 1.0000x reference)
import jax
import jax.numpy as jnp
from jax.experimental import pallas as pl
from jax.experimental.pallas import tpu as pltpu

B = 16384
INPUT_DIM = 128
EMBED_DIM = 16
CH = 2048
NCH = B // CH


def _fused_kernel(enc_hbm, floors_ref, emb_ref, w1_ref, w2_ref, b_ref,
                  out_hbm, enc_buf, out_buf, lsem, ssem):
    p = jax.lax.dot_general(
        emb_ref[...], w2_ref[...],
        dimension_numbers=(((1,), (1,)), ((), ())),
        preferred_element_type=jnp.float32,
    )
    pdiff = p[1:2, :] - p[0:1, :]
    base = p[0:1, :] + b_ref[...]

    loads = []
    for c in range(NCH):
        cp = pltpu.make_async_copy(
            enc_hbm.at[pl.ds(c * CH, CH), :], enc_buf.at[c], lsem.at[c])
        cp.start()
        loads.append(cp)

    stores = []
    for c in range(NCH):
        loads[c].wait()
        dense = jax.lax.dot_general(
            enc_buf[c], w1_ref[...],
            dimension_numbers=(((1,), (1,)), ((), ())),
            preferred_element_type=jnp.float32,
        )
        f = floors_ref[c, 0, :].astype(jnp.float32)[:, None]
        out_buf[c] = dense + f * pdiff + base
        st = pltpu.make_async_copy(
            out_buf.at[c], out_hbm.at[pl.ds(c * CH, CH), :], ssem.at[c])
        st.start()
        stores.append(st)

    for st in stores:
        st.wait()


def kernel(encodings, src_floors, emb_table, W, b):
    w1 = W[:, :INPUT_DIM]
    w2 = W[:, INPUT_DIM:]
    floors3 = src_floors.astype(jnp.int32).reshape(NCH, 1, CH)
    b2 = b.reshape(1, INPUT_DIM)
    return pl.pallas_call(
        _fused_kernel,
        in_specs=[
            pl.BlockSpec(memory_space=pl.ANY),
            pl.BlockSpec(memory_space=pltpu.MemorySpace.VMEM),
            pl.BlockSpec(memory_space=pltpu.MemorySpace.VMEM),
            pl.BlockSpec(memory_space=pltpu.MemorySpace.VMEM),
            pl.BlockSpec(memory_space=pltpu.MemorySpace.VMEM),
            pl.BlockSpec(memory_space=pltpu.MemorySpace.VMEM),
        ],
        out_specs=pl.BlockSpec(memory_space=pl.ANY),
        out_shape=jax.ShapeDtypeStruct((B, INPUT_DIM), jnp.float32),
        scratch_shapes=[
            pltpu.VMEM((NCH, CH, INPUT_DIM), jnp.float32),
            pltpu.VMEM((NCH, CH, INPUT_DIM), jnp.float32),
            pltpu.SemaphoreType.DMA((NCH,)),
            pltpu.SemaphoreType.DMA((NCH,)),
        ],
    )(encodings, floors3, emb_table, w1, w2, b2)
